# baseline (speedup 1.0000x reference)
import jax
import jax.numpy as jnp
from jax import lax
from jax.experimental import pallas as pl
from jax.experimental.pallas import tpu as pltpu

N = 10000
D_IN = 256
D_HID = 256
D_OUT = 64
BM = 400
NB = N // BM


def _body(x_ref, a_ref, w1_ref, w2_ref, w3_ref, out_ref, s_a, s_b):
    i = pl.program_id(0)
    phase = i // NB
    j = i % NB
    row = j * BM

    @pl.when(phase == 0)
    def _():
        s_a[pl.ds(row, BM), :] = jnp.dot(
            x_ref[...], w1_ref[...], preferred_element_type=jnp.float32)

    @pl.when(phase == 1)
    def _():
        acc = jnp.dot(a_ref[...], s_a[...],
                      preferred_element_type=jnp.float32)
        h = jnp.maximum(acc, 0.0)
        s_b[pl.ds(row, BM), :] = jnp.dot(
            h, w2_ref[...], preferred_element_type=jnp.float32)

    @pl.when(phase == 2)
    def _():
        acc = jnp.dot(a_ref[...], s_b[...],
                      preferred_element_type=jnp.float32)
        h = jnp.maximum(acc, 0.0)
        s_a[pl.ds(row, BM), :D_OUT] = jnp.dot(
            h, w3_ref[...], preferred_element_type=jnp.float32)

    @pl.when(phase == 3)
    def _():
        acc = jnp.dot(a_ref[...], s_a[:, :D_OUT],
                      preferred_element_type=jnp.float32)
        h = jnp.maximum(acc, 0.0)
        m = jnp.max(h, axis=-1, keepdims=True)
        e = jnp.exp(h - m)
        out_ref[...] = e / jnp.sum(e, axis=-1, keepdims=True)


def _x_idx(i):
    return (jnp.minimum(i, NB - 1), 0)


def _band_idx(i):
    return (jnp.maximum(i - NB, 0) % NB, 0)


def _out_idx(i):
    return (jnp.maximum(i - 3 * NB, 0), 0)


def kernel(input, adj, W1, W2, W3):
    return pl.pallas_call(
        _body,
        grid=(4 * NB,),
        in_specs=[
            pl.BlockSpec((BM, D_IN), _x_idx),
            pl.BlockSpec((BM, N), _band_idx),
            pl.BlockSpec((D_IN, D_HID), lambda i: (0, 0)),
            pl.BlockSpec((D_HID, D_HID), lambda i: (0, 0)),
            pl.BlockSpec((D_HID, D_OUT), lambda i: (0, 0)),
        ],
        out_specs=pl.BlockSpec((BM, D_OUT), _out_idx),
        out_shape=jax.ShapeDtypeStruct((N, D_OUT), jnp.float32),
        scratch_shapes=[
            pltpu.VMEM((N, D_HID), jnp.float32),
            pltpu.VMEM((N, D_HID), jnp.float32),
        ],
        compiler_params=pltpu.CompilerParams(
            dimension_semantics=("arbitrary",),
        ),
    )(input, adj, W1, W2, W3)


# speedup vs baseline: 1.0138x; 1.0138x over previous
import jax
import jax.numpy as jnp
from jax import lax
from jax.experimental import pallas as pl
from jax.experimental.pallas import tpu as pltpu

N = 10000
D_IN = 256
D_HID = 256
D_OUT = 64
BM = 400
NB = N // BM


def _mm_body(x_ref, w_ref, o_ref):
    o_ref[...] = jnp.dot(x_ref[...], w_ref[...],
                         preferred_element_type=jnp.float32)


def _layers_body(s1_ref, a_ref, w2_ref, w3_ref, out_ref, s_b, s_c):
    i = pl.program_id(0)
    j = i % NB
    layer = i // NB
    row = j * BM

    @pl.when(layer == 0)
    def _():
        acc = jnp.dot(a_ref[...], s1_ref[...],
                      preferred_element_type=jnp.float32)
        h = jnp.maximum(acc, 0.0)
        s_b[pl.ds(row, BM), :] = jnp.dot(
            h, w2_ref[...], preferred_element_type=jnp.float32)

    @pl.when(layer == 1)
    def _():
        acc = jnp.dot(a_ref[...], s_b[...],
                      preferred_element_type=jnp.float32)
        h = jnp.maximum(acc, 0.0)
        s_c[pl.ds(row, BM), :] = jnp.dot(
            h, w3_ref[...], preferred_element_type=jnp.float32)

    @pl.when(layer == 2)
    def _():
        acc = jnp.dot(a_ref[...], s_c[...],
                      preferred_element_type=jnp.float32)
        h = jnp.maximum(acc, 0.0)
        m = jnp.max(h, axis=-1, keepdims=True)
        e = jnp.exp(h - m)
        out_ref[...] = e / jnp.sum(e, axis=-1, keepdims=True)


def _band_idx(i):
    return (i % NB, 0)


def _out_idx(i):
    return (jnp.maximum(i - 2 * NB, 0), 0)


def kernel(input, adj, W1, W2, W3):
    s1 = pl.pallas_call(
        _mm_body,
        out_shape=jax.ShapeDtypeStruct((N, D_HID), jnp.float32),
    )(input, W1)

    return pl.pallas_call(
        _layers_body,
        grid=(3 * NB,),
        in_specs=[
            pl.BlockSpec((N, D_HID), lambda i: (0, 0)),
            pl.BlockSpec((BM, N), _band_idx),
            pl.BlockSpec((D_HID, D_HID), lambda i: (0, 0)),
            pl.BlockSpec((D_HID, D_OUT), lambda i: (0, 0)),
        ],
        out_specs=pl.BlockSpec((BM, D_OUT), _out_idx),
        out_shape=jax.ShapeDtypeStruct((N, D_OUT), jnp.float32),
        scratch_shapes=[
            pltpu.VMEM((N, D_HID), jnp.float32),
            pltpu.VMEM((N, D_OUT), jnp.float32),
        ],
        compiler_params=pltpu.CompilerParams(
            dimension_semantics=("arbitrary",),
        ),
    )(s1, adj, W2, W3)


# speedup vs baseline: 1.0185x; 1.0046x over previous
import jax
import jax.numpy as jnp
from jax import lax
from jax.experimental import pallas as pl
from jax.experimental.pallas import tpu as pltpu

N = 10000
D_IN = 256
D_HID = 256
D_OUT = 64
BM = 400
NB = N // BM


def _body(x_ref, a_ref, w1_ref, w2_ref, w3_ref, out_ref, s_a, s_b):
    i = pl.program_id(0)
    t = i - 1
    j = t % NB
    layer = t // NB
    row = j * BM

    @pl.when(i == 0)
    def _():
        def chunk(k, carry):
            r = k * BM
            s_a[pl.ds(r, BM), :] = jnp.dot(
                x_ref[pl.ds(r, BM), :], w1_ref[...],
                preferred_element_type=jnp.float32)
            return carry
        lax.fori_loop(0, NB, chunk, 0)

    @pl.when(layer == 0)
    def _():
        acc = jnp.dot(a_ref[...], s_a[...],
                      preferred_element_type=jnp.float32)
        h = jnp.maximum(acc, 0.0)
        s_b[pl.ds(row, BM), :] = jnp.dot(
            h, w2_ref[...], preferred_element_type=jnp.float32)

    @pl.when(layer == 1)
    def _():
        acc = jnp.dot(a_ref[...], s_b[...],
                      preferred_element_type=jnp.float32)
        h = jnp.maximum(acc, 0.0)
        s_a[pl.ds(row, BM), :D_OUT] = jnp.dot(
            h, w3_ref[...], preferred_element_type=jnp.float32)

    @pl.when(layer == 2)
    def _():
        acc = jnp.dot(a_ref[...], s_a[:, :D_OUT],
                      preferred_element_type=jnp.float32)
        h = jnp.maximum(acc, 0.0)
        m = jnp.max(h, axis=-1, keepdims=True)
        e = jnp.exp(h - m)
        out_ref[...] = e / jnp.sum(e, axis=-1, keepdims=True)


def _band_idx(i):
    return (jnp.maximum(i - 1, 0) % NB, 0)


def _out_idx(i):
    return (jnp.maximum(i - 1 - 2 * NB, 0), 0)


def kernel(input, adj, W1, W2, W3):
    return pl.pallas_call(
        _body,
        grid=(1 + 3 * NB,),
        in_specs=[
            pl.BlockSpec((N, D_IN), lambda i: (0, 0)),
            pl.BlockSpec((BM, N), _band_idx),
            pl.BlockSpec((D_IN, D_HID), lambda i: (0, 0)),
            pl.BlockSpec((D_HID, D_HID), lambda i: (0, 0)),
            pl.BlockSpec((D_HID, D_OUT), lambda i: (0, 0)),
        ],
        out_specs=pl.BlockSpec((BM, D_OUT), _out_idx),
        out_shape=jax.ShapeDtypeStruct((N, D_OUT), jnp.float32),
        scratch_shapes=[
            pltpu.VMEM((N, D_HID), jnp.float32),
            pltpu.VMEM((N, D_HID), jnp.float32),
        ],
        compiler_params=pltpu.CompilerParams(
            dimension_semantics=("arbitrary",),
            vmem_limit_bytes=128 * 1024 * 1024,
        ),
    )(input, adj, W1, W2, W3)
